# baseline (speedup 1.0000x reference)
import functools

import jax
import jax.numpy as jnp
from jax import lax
from jax.experimental import pallas as pl
from jax.experimental.pallas import tpu as pltpu
from jax.experimental.pallas import tpu_sc as plsc

P = 128
D = 24576
B = 16 * 128
NC, NS = 2, 16
HALF = D // NC
RPT = B // NS
RLD = P // NS

_mesh = plsc.VectorSubcoreMesh(core_axis_name="c", subcore_axis_name="s")


@functools.partial(
    pl.kernel,
    mesh=_mesh,
    out_type=jax.ShapeDtypeStruct((B, NC, HALF), jnp.float32),
    scratch_types=[
        pltpu.VMEM_SHARED((P, HALF), jnp.float32),
        pltpu.VMEM((RPT,), jnp.int32),
        pltpu.SemaphoreType.DMA,
    ],
)
def _gather(idx_hbm, table_hbm, out_hbm, cache, idx_v, sem):
    c = lax.axis_index("c")
    s = lax.axis_index("s")
    pltpu.sync_copy(
        table_hbm.at[pl.ds(s * RLD, RLD), pl.ds(c * HALF, HALF)],
        cache.at[pl.ds(s * RLD, RLD)],
    )
    base = s * RPT
    pltpu.sync_copy(idx_hbm.at[pl.ds(base, RPT)], idx_v)
    plsc.subcore_barrier()

    def fire(g, carry):
        vec = idx_v[pl.ds(g * 16, 16)]
        for k in range(16):
            pltpu.async_copy(
                cache.at[pl.ds(vec[k], 1)],
                out_hbm.at[pl.ds(base + g * 16 + k, 1), c],
                sem,
            )
        return carry

    lax.fori_loop(0, RPT // 16, fire, 0)

    def drain(i, carry):
        pltpu.make_async_copy(
            cache.at[pl.ds(0, 1)],
            out_hbm.at[pl.ds(base, 1), c],
            sem,
        ).wait()
        return carry

    lax.fori_loop(0, RPT, drain, 0)


def kernel(prefix, table):
    idx = prefix.reshape(B).astype(jnp.int32)
    out = _gather(idx, table)
    return out.reshape(prefix.shape[0], prefix.shape[1], D)


# speedup vs baseline: 2.3388x; 2.3388x over previous
import functools

import jax
import jax.numpy as jnp
from jax import lax
from jax.experimental import pallas as pl
from jax.experimental.pallas import tpu as pltpu
from jax.experimental.pallas import tpu_sc as plsc

P = 128
D = 24576
B = 16 * 128
NC, NS = 2, 16
NW = NC * NS
DC = D // NW
GRP = B // 16
LAG = 4

_mesh = plsc.VectorSubcoreMesh(core_axis_name="c", subcore_axis_name="s")


@functools.partial(
    pl.kernel,
    mesh=_mesh,
    out_type=jax.ShapeDtypeStruct((B, NW, DC), jnp.float32),
    scratch_types=[
        pltpu.VMEM((P, DC), jnp.float32),
        pltpu.VMEM((B,), jnp.int32),
        pltpu.SemaphoreType.DMA,
    ],
)
def _gather(idx_hbm, table_hbm, out_hbm, cache, idx_v, sem):
    w = lax.axis_index("s") * NC + lax.axis_index("c")
    pltpu.sync_copy(table_hbm.at[:, pl.ds(w * DC, DC)], cache)
    pltpu.sync_copy(idx_hbm, idx_v)

    def fire(g):
        vec = idx_v[pl.ds(g * 16, 16)]
        for k in range(16):
            pltpu.async_copy(
                cache.at[pl.ds(vec[k], 1)],
                out_hbm.at[pl.ds(g * 16 + k, 1), w],
                sem,
            )

    def drain(g):
        for k in range(16):
            pltpu.make_async_copy(
                cache.at[pl.ds(0, 1)],
                out_hbm.at[pl.ds(g * 16 + k, 1), w],
                sem,
            ).wait()

    for g in range(LAG):
        fire(g)

    def body(g, carry):
        fire(g)
        drain(g - LAG)
        return carry

    lax.fori_loop(LAG, GRP, body, 0)

    def tail(g, carry):
        drain(g)
        return carry

    lax.fori_loop(GRP - LAG, GRP, tail, 0)


def kernel(prefix, table):
    idx = prefix.reshape(B).astype(jnp.int32)
    out = _gather(idx, table)
    return out.reshape(prefix.shape[0], prefix.shape[1], D)
